# baseline (speedup 1.0000x reference)
import jax
import jax.numpy as jnp
from jax import lax
from jax.experimental import pallas as pl
from jax.experimental.pallas import tpu as pltpu
from jax.experimental.pallas import tpu_sc as plsc

_NUM_CORES = 2
_CHUNKS = 4


def _scs_copy_body(pos_hbm, out_hbm, buf0, buf1, insem, outsem):
    rows = pos_hbm.shape[0] // _NUM_CORES
    chunk = rows // _CHUNKS
    base = lax.axis_index("c") * rows
    bufs = (buf0, buf1)

    def start_in(k):
        return pltpu.async_copy(
            pos_hbm.at[pl.ds(base + k * chunk, chunk)], bufs[k % 2], insem)

    def start_out(k):
        return pltpu.async_copy(
            bufs[k % 2], out_hbm.at[pl.ds(base + k * chunk, chunk)], outsem)

    ins = [None] * _CHUNKS
    outs = [None] * _CHUNKS
    ins[0] = start_in(0)
    ins[1] = start_in(1)
    ins[0].wait()
    outs[0] = start_out(0)
    for k in range(1, _CHUNKS):
        ins[k].wait()
        outs[k] = start_out(k)
        if k + 1 < _CHUNKS:
            outs[k - 1].wait()
            ins[k + 1] = start_in(k + 1)
    outs[_CHUNKS - 2].wait()
    outs[_CHUNKS - 1].wait()


def kernel(x, pos_emb):
    seq_len = x.shape[1]
    d = pos_emb.shape[1]
    table = pos_emb[:seq_len]
    chunk_rows = seq_len // _NUM_CORES // _CHUNKS
    mesh = plsc.ScalarSubcoreMesh(axis_name="c", num_cores=_NUM_CORES)
    out = pl.kernel(
        _scs_copy_body,
        mesh=mesh,
        out_type=jax.ShapeDtypeStruct((seq_len, d), pos_emb.dtype),
        scratch_types=[
            pltpu.MemorySpace.VMEM_SHARED((chunk_rows, d), pos_emb.dtype),
            pltpu.MemorySpace.VMEM_SHARED((chunk_rows, d), pos_emb.dtype),
            pltpu.SemaphoreType.DMA,
            pltpu.SemaphoreType.DMA,
        ],
    )(table)
    return out[None]


# speedup vs baseline: 1.1045x; 1.1045x over previous
import jax
import jax.numpy as jnp
from jax import lax
from jax.experimental import pallas as pl
from jax.experimental.pallas import tpu as pltpu
from jax.experimental.pallas import tpu_sc as plsc

_NUM_CORES = 2
_NUM_SUBCORES = 16
_NUM_WORKERS = _NUM_CORES * _NUM_SUBCORES


def _sc_copy_body(pos_hbm, out_hbm, buf0, buf1, insem, outsem):
    rows = pos_hbm.shape[0] // _NUM_WORKERS
    half = rows // 2
    wid = lax.axis_index("s") * _NUM_CORES + lax.axis_index("c")
    base = wid * rows

    in0 = pltpu.async_copy(pos_hbm.at[pl.ds(base, half)], buf0, insem)
    in1 = pltpu.async_copy(pos_hbm.at[pl.ds(base + half, half)], buf1, insem)
    in0.wait()
    out0 = pltpu.async_copy(buf0, out_hbm.at[pl.ds(base, half)], outsem)
    in1.wait()
    out1 = pltpu.async_copy(buf1, out_hbm.at[pl.ds(base + half, half)], outsem)
    out0.wait()
    out1.wait()


def kernel(x, pos_emb):
    seq_len = x.shape[1]
    d = pos_emb.shape[1]
    table = pos_emb[:seq_len]
    half_rows = seq_len // _NUM_WORKERS // 2
    mesh = plsc.VectorSubcoreMesh(core_axis_name="c", subcore_axis_name="s")
    out = pl.kernel(
        _sc_copy_body,
        mesh=mesh,
        out_type=jax.ShapeDtypeStruct((seq_len, d), pos_emb.dtype),
        scratch_types=[
            pltpu.VMEM((half_rows, d), pos_emb.dtype),
            pltpu.VMEM((half_rows, d), pos_emb.dtype),
            pltpu.SemaphoreType.DMA,
            pltpu.SemaphoreType.DMA,
        ],
    )(table)
    return out[None]


# speedup vs baseline: 1.1173x; 1.0116x over previous
import jax
import jax.numpy as jnp
from jax import lax
from jax.experimental import pallas as pl
from jax.experimental.pallas import tpu as pltpu
from jax.experimental.pallas import tpu_sc as plsc

_NUM_CORES = 2
_NUM_SUBCORES = 16
_NUM_WORKERS = _NUM_CORES * _NUM_SUBCORES


def _sc_lookup_body(pos_hbm, out_hbm, buf, sem):
    rows = pos_hbm.shape[0] // _NUM_WORKERS
    wid = lax.axis_index("s") * _NUM_CORES + lax.axis_index("c")
    base = wid * rows
    pltpu.async_copy(pos_hbm.at[pl.ds(base, rows)], buf, sem).wait()
    pltpu.async_copy(buf, out_hbm.at[pl.ds(base, rows)], sem).wait()


def kernel(x, pos_emb):
    seq_len = x.shape[1]
    d = pos_emb.shape[1]
    table = pos_emb[:seq_len]
    mesh = plsc.VectorSubcoreMesh(core_axis_name="c", subcore_axis_name="s")
    out = pl.kernel(
        _sc_lookup_body,
        mesh=mesh,
        out_type=jax.ShapeDtypeStruct((seq_len, d), pos_emb.dtype),
        scratch_types=[
            pltpu.VMEM((seq_len // _NUM_WORKERS, d), pos_emb.dtype),
            pltpu.SemaphoreType.DMA,
        ],
    )(table)
    return out[None]
